# Initial kernel scaffold; baseline (speedup 1.0000x reference)
import jax
import jax.numpy as jnp
from jax.experimental import pallas as pl


def kernel(residue_index, asym_id, residue_embed, chain_embed):
    raise NotImplementedError("write your pallas kernel here")


# speedup vs baseline: 1.9898x; 1.9898x over previous
import functools

import jax
import jax.numpy as jnp
from jax import lax
from jax.experimental import pallas as pl
from jax.experimental.pallas import tpu as pltpu
from jax.experimental.pallas import tpu_sc as plsc

HIDDEN_HALF = 1024
MAX_RES = 4096
MAX_CHAINS = 64
N_TOKENS = 16384

_info = plsc.get_sparse_core_info()
NC = _info.num_cores
NS = _info.num_subcores
L = _info.num_lanes
NW = NC * NS
B_PER_W = N_TOKENS // NW
CHUNK = 32
N_CHUNKS = B_PER_W // CHUNK

_mesh = plsc.VectorSubcoreMesh(core_axis_name="c", subcore_axis_name="s")


@functools.partial(
    pl.kernel,
    mesh=_mesh,
    out_type=jax.ShapeDtypeStruct((N_TOKENS, 2 * HIDDEN_HALF), jnp.float32),
    scratch_types=[
        pltpu.VMEM((B_PER_W,), jnp.int32),
        pltpu.VMEM((B_PER_W,), jnp.int32),
        pltpu.VMEM((CHUNK, HIDDEN_HALF), jnp.float32),
        pltpu.VMEM((CHUNK, HIDDEN_HALF), jnp.float32),
        pltpu.SemaphoreType.DMA,
        pltpu.SemaphoreType.DMA,
    ],
)
def _embed_kernel(res_idx_hbm, chain_idx_hbm, res_tab_hbm, chain_tab_hbm,
                  out_hbm, ridx_v, cidx_v, rbuf, cbuf, gsem, wsem):
    wid = lax.axis_index("s") * NC + lax.axis_index("c")
    base = wid * B_PER_W

    pltpu.sync_copy(res_idx_hbm.at[pl.ds(base, B_PER_W)], ridx_v)
    pltpu.sync_copy(chain_idx_hbm.at[pl.ds(base, B_PER_W)], cidx_v)

    def _clip(i, carry):
        sl = pl.ds(i * L, L)
        r = ridx_v[sl]
        ridx_v[sl] = jnp.clip(r - 1, 0, MAX_RES - 1)
        c = cidx_v[sl]
        cidx_v[sl] = jnp.clip(c, 0, MAX_CHAINS - 1)
        return carry

    lax.fori_loop(0, B_PER_W // L, _clip, 0)

    for c in range(N_CHUNKS):
        row0 = base + c * CHUNK
        isl = pl.ds(c * CHUNK, CHUNK)
        pltpu.async_copy(res_tab_hbm.at[ridx_v.at[isl]], rbuf, gsem).wait()
        wr = pltpu.async_copy(
            rbuf, out_hbm.at[pl.ds(row0, CHUNK), pl.ds(0, HIDDEN_HALF)], wsem)
        pltpu.async_copy(chain_tab_hbm.at[cidx_v.at[isl]], cbuf, gsem).wait()
        wc = pltpu.async_copy(
            cbuf, out_hbm.at[pl.ds(row0, CHUNK), pl.ds(HIDDEN_HALF, HIDDEN_HALF)],
            wsem)
        wr.wait()
        wc.wait()


def kernel(residue_index, asym_id, residue_embed, chain_embed):
    return _embed_kernel(residue_index.astype(jnp.int32),
                         asym_id.astype(jnp.int32),
                         residue_embed, chain_embed)


# speedup vs baseline: 2.0294x; 1.0199x over previous
import functools

import jax
import jax.numpy as jnp
from jax import lax
from jax.experimental import pallas as pl
from jax.experimental.pallas import tpu as pltpu
from jax.experimental.pallas import tpu_sc as plsc

HIDDEN_HALF = 1024
MAX_RES = 4096
MAX_CHAINS = 64
N_TOKENS = 16384

_info = plsc.get_sparse_core_info()
NC = _info.num_cores
NS = _info.num_subcores
L = _info.num_lanes
B_PER_T = N_TOKENS // NS
CHUNK = 32
N_CHUNKS = B_PER_T // CHUNK

_mesh = plsc.VectorSubcoreMesh(core_axis_name="c", subcore_axis_name="s")


@functools.partial(
    pl.kernel,
    mesh=_mesh,
    out_type=jax.ShapeDtypeStruct((N_TOKENS, 2 * HIDDEN_HALF), jnp.float32),
    scratch_types=[
        pltpu.VMEM((B_PER_T,), jnp.int32),
        pltpu.VMEM((CHUNK, HIDDEN_HALF), jnp.float32),
        pltpu.VMEM((CHUNK, HIDDEN_HALF), jnp.float32),
        pltpu.SemaphoreType.DMA,
        pltpu.SemaphoreType.DMA,
    ],
)
def _embed_kernel(res_idx_hbm, chain_idx_hbm, res_tab_hbm, chain_tab_hbm,
                  out_hbm, idx_v, buf0, buf1, gsem, wsem):
    core = lax.axis_index("c")
    sub = lax.axis_index("s")
    base = sub * B_PER_T
    bufs = (buf0, buf1)

    def run_table(idx_hbm, tab_hbm, col0, lo, hi, shift):
        pltpu.sync_copy(idx_hbm.at[pl.ds(base, B_PER_T)], idx_v)

        def _clip(i, carry):
            sl = pl.ds(i * L, L)
            idx_v[sl] = jnp.clip(idx_v[sl] - shift, lo, hi)
            return carry

        lax.fori_loop(0, B_PER_T // L, _clip, 0)

        g = [None, None]
        w = [None, None]
        g[0] = pltpu.async_copy(
            tab_hbm.at[idx_v.at[pl.ds(0, CHUNK)]], bufs[0], gsem)
        for c in range(N_CHUNKS):
            cur = c & 1
            nxt = 1 - cur
            g[cur].wait()
            if c + 1 < N_CHUNKS:
                if w[nxt] is not None:
                    w[nxt].wait()
                g[nxt] = pltpu.async_copy(
                    tab_hbm.at[idx_v.at[pl.ds((c + 1) * CHUNK, CHUNK)]],
                    bufs[nxt], gsem)
            w[cur] = pltpu.async_copy(
                bufs[cur],
                out_hbm.at[pl.ds(base + c * CHUNK, CHUNK),
                           pl.ds(col0, HIDDEN_HALF)],
                wsem)
        w[0].wait()
        w[1].wait()

    @pl.when(core == 0)
    def _():
        run_table(res_idx_hbm, res_tab_hbm, 0, 0, MAX_RES - 1, 1)

    @pl.when(core == 1)
    def _():
        run_table(chain_idx_hbm, chain_tab_hbm, HIDDEN_HALF, 0,
                  MAX_CHAINS - 1, 0)


def kernel(residue_index, asym_id, residue_embed, chain_embed):
    return _embed_kernel(residue_index.astype(jnp.int32),
                         asym_id.astype(jnp.int32),
                         residue_embed, chain_embed)
